# baseline (speedup 1.0000x reference)
import functools
import jax
import jax.numpy as jnp
from jax import lax
from jax.experimental import pallas as pl
from jax.experimental.pallas import tpu as pltpu, tpu_sc as plsc

_R = 1024
_C = 26
_SIZE = 1000
_TR = _SIZE // 8
_NSLAB = _C * _TR
_SLAB = 8192
_CHUNK = 2 * _SLAB
_NBUF = 4


def _divmod125(s):
    c = (s * 8389) >> 20
    return c, s - c * _TR


def _slab_compute(idx_v, buf, off, c_local, tr):
    k0 = tr * 8

    def _tc_body(tc, _):
        base = c_local * _R + tc * 128
        ivs = [idx_v[pl.ds(base + g * 16, 16)] for g in range(8)]
        for ks in range(8):
            kvec = jnp.full((16,), k0 + ks, jnp.int32)
            for g in range(8):
                buf[pl.ds(off + tc * 1024 + ks * 128 + g * 16, 16)] = (
                    ivs[g] == kvec
                ).astype(jnp.float32)
        return _

    lax.fori_loop(0, 8, _tc_body, 0)


def _advance(c, tr):
    wrap = (tr + 1 >= _TR).astype(jnp.int32)
    return c + wrap, (tr + 1) - wrap * _TR


def _sc_body(idx_hbm, out_hbm, idx_v, *bufs_and_sems):
    bufs = bufs_and_sems[:_NBUF]
    sems = bufs_and_sems[_NBUF:]
    nc = 2
    w = lax.axis_index("s") * nc + lax.axis_index("c")

    base = jnp.where(w < 25, 102 * w, 100 * w + 50)
    cb = base // 2
    c0, tr0 = _divmod125(base)

    pltpu.sync_copy(idx_hbm.at[pl.ds(c0 * _R, 2 * _R)], idx_v)

    def _start(buf, sem, q):
        return pltpu.async_copy(
            buf, out_hbm.at[pl.ds(q * _CHUNK, _CHUNK)], sem
        )

    def _drain(buf, sem):
        pltpu.make_async_copy(buf, out_hbm.at[pl.ds(0, _CHUNK)], sem).wait()

    def _chunk_compute(buf, c, tr):
        _slab_compute(idx_v, buf, 0, c - c0, tr)
        c, tr = _advance(c, tr)
        _slab_compute(idx_v, buf, _SLAB, c - c0, tr)
        return _advance(c, tr)

    c, tr = c0, tr0
    for b in range(_NBUF):
        c, tr = _chunk_compute(bufs[b], c, tr)
        _start(bufs[b], sems[b], cb + b)

    def _loop_body(j, carry):
        c, tr = carry
        for b in range(_NBUF):
            _drain(bufs[b], sems[b])
            c, tr = _chunk_compute(bufs[b], c, tr)
            _start(bufs[b], sems[b], cb + 4 * j + b)
        return c, tr

    c, tr = lax.fori_loop(1, 12, _loop_body, (c, tr))

    for b in range(2):
        _drain(bufs[b], sems[b])
        c, tr = _chunk_compute(bufs[b], c, tr)
        _start(bufs[b], sems[b], cb + 48 + b)

    @pl.when(w < 25)
    def _extra():
        _drain(bufs[2], sems[2])
        _chunk_compute(bufs[2], c, tr)
        _start(bufs[2], sems[2], cb + 50)

    for b in range(_NBUF):
        _drain(bufs[b], sems[b])


_sc_onehot = functools.partial(
    pl.kernel,
    mesh=plsc.VectorSubcoreMesh(core_axis_name="c", subcore_axis_name="s"),
    out_type=jax.ShapeDtypeStruct((_R * _C * _SIZE,), jnp.float32),
    compiler_params=pltpu.CompilerParams(needs_layout_passes=False),
    scratch_types=[
        pltpu.VMEM((2 * _R,), jnp.int32),
        *([pltpu.VMEM((_CHUNK,), jnp.float32)] * _NBUF),
        *([pltpu.SemaphoreType.DMA] * _NBUF),
    ],
)(_sc_body)


def kernel(x, size):
    del size
    idx_t = x.astype(jnp.int32).T.reshape(_C * _R)
    idx_t = jnp.pad(idx_t, (0, _R))
    out = _sc_onehot(idx_t)
    return (
        out.reshape(_C, _TR, 8, 8, 128)
        .transpose(2, 4, 0, 1, 3)
        .reshape(_R, _C, _SIZE)
    )


# speedup vs baseline: 1.2171x; 1.2171x over previous
import functools
import jax
import jax.numpy as jnp
from jax import lax
from jax.experimental import pallas as pl
from jax.experimental.pallas import tpu as pltpu, tpu_sc as plsc

_R = 1024
_C = 26
_SIZE = 1000
_TR = _SIZE // 8
_NSLAB = _C * _TR
_SLAB = 8192
_CHUNK = 2 * _SLAB
_NBUF = 2


def _divmod125(s):
    c = (s * 8389) >> 20
    return c, s - c * _TR


def _slab_compute(idx_v, buf, off, c_local, tr):
    k0 = tr * 8

    def _tc_body(tc, _):
        base = c_local * _R + tc * 128
        ivs = [idx_v[pl.ds(base + g * 16, 16)] for g in range(8)]
        for ks in range(8):
            kvec = jnp.full((16,), k0 + ks, jnp.int32)
            for g in range(8):
                buf[pl.ds(off + tc * 1024 + ks * 128 + g * 16, 16)] = (
                    ivs[g] == kvec
                ).astype(jnp.float32)
        return _

    lax.fori_loop(0, 8, _tc_body, 0)


def _advance(c, tr):
    wrap = (tr + 1 >= _TR).astype(jnp.int32)
    return c + wrap, (tr + 1) - wrap * _TR


def _sc_body(idx_hbm, out_hbm, idx_v, *bufs_and_sems):
    bufs = bufs_and_sems[:_NBUF]
    sems = bufs_and_sems[_NBUF:]
    nc = 2
    w = lax.axis_index("s") * nc + lax.axis_index("c")

    base = jnp.where(w < 25, 102 * w, 100 * w + 50)
    cb = base // 2
    c0, tr0 = _divmod125(base)

    pltpu.sync_copy(idx_hbm.at[pl.ds(c0 * _R, 2 * _R)], idx_v)

    def _start(buf, sem, q):
        return pltpu.async_copy(
            buf, out_hbm.at[pl.ds(q * _CHUNK, _CHUNK)], sem
        )

    def _drain(buf, sem):
        pltpu.make_async_copy(buf, out_hbm.at[pl.ds(0, _CHUNK)], sem).wait()

    def _chunk_compute(buf, c, tr):
        _slab_compute(idx_v, buf, 0, c - c0, tr)
        c, tr = _advance(c, tr)
        _slab_compute(idx_v, buf, _SLAB, c - c0, tr)
        return _advance(c, tr)

    c, tr = c0, tr0
    for b in range(_NBUF):
        c, tr = _chunk_compute(bufs[b], c, tr)
        _start(bufs[b], sems[b], cb + b)

    def _loop_body(j, carry):
        c, tr = carry
        for b in range(_NBUF):
            _drain(bufs[b], sems[b])
            c, tr = _chunk_compute(bufs[b], c, tr)
            _start(bufs[b], sems[b], cb + _NBUF * j + b)
        return c, tr

    c, tr = lax.fori_loop(1, 24, _loop_body, (c, tr))

    for b in range(2):
        _drain(bufs[b], sems[b])
        c, tr = _chunk_compute(bufs[b], c, tr)
        _start(bufs[b], sems[b], cb + 48 + b)

    @pl.when(w < 25)
    def _extra():
        _drain(bufs[0], sems[0])
        _chunk_compute(bufs[0], c, tr)
        _start(bufs[0], sems[0], cb + 50)

    for b in range(_NBUF):
        _drain(bufs[b], sems[b])


_sc_onehot = functools.partial(
    pl.kernel,
    mesh=plsc.VectorSubcoreMesh(core_axis_name="c", subcore_axis_name="s"),
    out_type=jax.ShapeDtypeStruct((_R * _C * _SIZE,), jnp.float32),
    compiler_params=pltpu.CompilerParams(needs_layout_passes=False),
    scratch_types=[
        pltpu.VMEM((2 * _R,), jnp.int32),
        *([pltpu.VMEM((_CHUNK,), jnp.float32)] * _NBUF),
        *([pltpu.SemaphoreType.DMA] * _NBUF),
    ],
)(_sc_body)


def kernel(x, size):
    del size
    idx_t = x.astype(jnp.int32).T.reshape(_C * _R)
    idx_t = jnp.pad(idx_t, (0, _R))
    out = _sc_onehot(idx_t)
    return (
        out.reshape(_C, _TR, 8, 8, 128)
        .transpose(2, 4, 0, 1, 3)
        .reshape(_R, _C, _SIZE)
    )
